# baseline (speedup 1.0000x reference)
import jax
import jax.numpy as jnp
from jax.experimental import pallas as pl
from jax.experimental.pallas import tpu as pltpu

NC = 19
C = 256
Q = 2975
BS = 8
H = 128
HB = 64
W = 128
NRB = H // HB
NFB = BS * NRB
HWB = HB * W
INV_T = 5.0
CB = 16
NCB = C // CB
NBUF = 4
NSTEPS = NFB + 1


def _fused_kernel(fea_ref, res_ref, q_hbm, out_ref,
                  sums, cnt, term_acc, qbuf, qsem):
    i = pl.program_id(0)

    @pl.when(i == 0)
    def _init():
        sums[...] = jnp.zeros_like(sums)
        cnt[...] = jnp.zeros_like(cnt)
        term_acc[...] = jnp.zeros_like(term_acc)

    @pl.when(i < NFB)
    def _phase1():
        resb = res_ref[0]
        feab = fea_ref[0]

        maxv = resb[0:1]
        idx = jnp.zeros((1, HB, W), jnp.int32)
        for k in range(1, NC):
            row = resb[k:k + 1]
            upd = row > maxv
            maxv = jnp.where(upd, row, maxv)
            idx = jnp.where(upd, jnp.int32(k), idx)

        idx_flat = idx.reshape(1, HWB)
        cls_iota = jax.lax.broadcasted_iota(jnp.int32, (NC, HWB), 0)
        onehot = (idx_flat == cls_iota).astype(jnp.float32)
        fea_flat = feab.reshape(C, HWB)

        sums[...] += jax.lax.dot_general(
            fea_flat, onehot, (((1,), (1,)), ((), ())),
            preferred_element_type=jnp.float32)
        cnt[...] += jnp.sum(onehot, axis=1, keepdims=True)

    @pl.when(i == NSTEPS - 1)
    def _phase2():
        def _copy(k, slot):
            return pltpu.make_async_copy(
                q_hbm.at[:, pl.ds(k * CB, CB), :], qbuf.at[slot],
                qsem.at[slot])

        for k in range(NBUF - 1):
            _copy(k, k % NBUF).start()

        for k in range(NCB):
            slot = k % NBUF
            _copy(k, slot).wait()
            nk = k + NBUF - 1
            if nk < NCB:
                _copy(nk, nk % NBUF).start()

            row0 = k * CB
            qc = qbuf[slot]
            qsum_c = jnp.sum(qc, axis=0)

            for cls in range(NC):
                full_col = sums[:, cls:cls + 1]
                n2 = jnp.sum(full_col * full_col, axis=0,
                             keepdims=True)
                col = sums[pl.ds(row0, CB), cls:cls + 1]
                s_col = col * jax.lax.rsqrt(n2) * INV_T

                qb = qc[cls]
                posv = s_col * qb
                negv = s_col * (qsum_c - qb)
                m = jnp.max(jnp.maximum(posv, negv), axis=1,
                            keepdims=True)
                z = (jnp.sum(jnp.exp(posv - m), axis=1, keepdims=True)
                     + jnp.sum(jnp.exp(negv - m), axis=1, keepdims=True))
                lse = m + jnp.log(z)
                l0 = posv[:, 0:1]
                partial = jnp.sum(lse - l0, axis=0,
                                  keepdims=True)
                term_acc[:, cls:cls + 1] += partial

        loss = jnp.zeros((1, 1), jnp.float32)
        for cls in range(NC):
            cntv = cnt[cls:cls + 1, 0:1]
            term = term_acc[:, cls:cls + 1] / C
            loss = loss + jnp.where(cntv > 0, term, 0.0)
        out_ref[...] = loss


def kernel(fea, res, queues):
    out = pl.pallas_call(
        _fused_kernel,
        grid=(NSTEPS,),
        in_specs=[
            pl.BlockSpec((1, C, HB, W),
                         lambda i: (jnp.minimum(i // NRB, BS - 1), 0,
                                    jnp.where(i < NFB, i % NRB, NRB - 1), 0)),
            pl.BlockSpec((1, NC, HB, W),
                         lambda i: (jnp.minimum(i // NRB, BS - 1), 0,
                                    jnp.where(i < NFB, i % NRB, NRB - 1), 0)),
            pl.BlockSpec(memory_space=pl.ANY),
        ],
        out_specs=pl.BlockSpec((1, 1), lambda i: (0, 0)),
        out_shape=jax.ShapeDtypeStruct((1, 1), jnp.float32),
        scratch_shapes=[
            pltpu.VMEM((C, NC), jnp.float32),
            pltpu.VMEM((NC, 1), jnp.float32),
            pltpu.VMEM((1, NC), jnp.float32),
            pltpu.VMEM((NBUF, NC, CB, Q), jnp.float32),
            pltpu.SemaphoreType.DMA((NBUF,)),
        ],
    )(fea, res, queues)

    return out[0, 0]


# speedup vs baseline: 1.1679x; 1.1679x over previous
import jax
import jax.numpy as jnp
from jax.experimental import pallas as pl
from jax.experimental.pallas import tpu as pltpu

NC = 19
C = 256
Q = 2975
BS = 8
H = 128
HB = 64
W = 128
NRB = H // HB
NFB = BS * NRB
HWB = HB * W
INV_T = 5.0
CB = 16
NCB = C // CB
NSTEPS = NFB + NCB


def _fused_kernel(fea_ref, res_ref, q_ref, out_ref,
                  sums, cnt, term_acc):
    i = pl.program_id(0)

    @pl.when(i == 0)
    def _init():
        sums[...] = jnp.zeros_like(sums)
        cnt[...] = jnp.zeros_like(cnt)
        term_acc[...] = jnp.zeros_like(term_acc)

    @pl.when(i < NFB)
    def _phase1():
        resb = res_ref[0]
        feab = fea_ref[0]

        maxv = resb[0:1]
        idx = jnp.zeros((1, HB, W), jnp.int32)
        for k in range(1, NC):
            row = resb[k:k + 1]
            upd = row > maxv
            maxv = jnp.where(upd, row, maxv)
            idx = jnp.where(upd, jnp.int32(k), idx)

        idx_flat = idx.reshape(1, HWB)
        cls_iota = jax.lax.broadcasted_iota(jnp.int32, (NC, HWB), 0)
        onehot = (idx_flat == cls_iota).astype(jnp.float32)
        fea_flat = feab.reshape(C, HWB)

        sums[...] += jax.lax.dot_general(
            fea_flat, onehot, (((1,), (1,)), ((), ())),
            preferred_element_type=jnp.float32)
        cnt[...] += jnp.sum(onehot, axis=1, keepdims=True)

    @pl.when(i >= NFB)
    def _phase2():
        row0 = (i - NFB) * CB
        qc = q_ref[...]
        qsum_c = jnp.sum(qc, axis=0)

        for cls in range(NC):
            full_col = sums[:, cls:cls + 1]
            n2 = jnp.sum(full_col * full_col, axis=0,
                         keepdims=True)
            col = sums[pl.ds(row0, CB), cls:cls + 1]
            s_col = col * jax.lax.rsqrt(n2) * INV_T

            qb = qc[cls]
            posv = s_col * qb
            negv = s_col * (qsum_c - qb)
            m = jnp.max(jnp.maximum(posv, negv), axis=1,
                        keepdims=True)
            z = (jnp.sum(jnp.exp(posv - m), axis=1, keepdims=True)
                 + jnp.sum(jnp.exp(negv - m), axis=1, keepdims=True))
            lse = m + jnp.log(z)
            l0 = posv[:, 0:1]
            partial = jnp.sum(lse - l0, axis=0, keepdims=True)
            term_acc[:, cls:cls + 1] += partial

    @pl.when(i == NSTEPS - 1)
    def _finalize():
        loss = jnp.zeros((1, 1), jnp.float32)
        for cls in range(NC):
            cntv = cnt[cls:cls + 1, 0:1]
            term = term_acc[:, cls:cls + 1] / C
            loss = loss + jnp.where(cntv > 0, term, 0.0)
        out_ref[...] = loss


def kernel(fea, res, queues):
    out = pl.pallas_call(
        _fused_kernel,
        grid=(NSTEPS,),
        in_specs=[
            pl.BlockSpec((1, C, HB, W),
                         lambda i: (jnp.minimum(i // NRB, BS - 1), 0,
                                    jnp.where(i < NFB, i % NRB, NRB - 1), 0)),
            pl.BlockSpec((1, NC, HB, W),
                         lambda i: (jnp.minimum(i // NRB, BS - 1), 0,
                                    jnp.where(i < NFB, i % NRB, NRB - 1), 0)),
            pl.BlockSpec((NC, CB, Q),
                         lambda i: (0, jnp.clip(i - NFB, 0, NCB - 1), 0)),
        ],
        out_specs=pl.BlockSpec((1, 1), lambda i: (0, 0)),
        out_shape=jax.ShapeDtypeStruct((1, 1), jnp.float32),
        scratch_shapes=[
            pltpu.VMEM((C, NC), jnp.float32),
            pltpu.VMEM((NC, 1), jnp.float32),
            pltpu.VMEM((1, NC), jnp.float32),
        ],
    )(fea, res, queues)

    return out[0, 0]
